# baseline (speedup 1.0000x reference)
import functools

import jax
import jax.numpy as jnp
from jax import lax
from jax.experimental import pallas as pl
from jax.experimental.pallas import tpu as pltpu
from jax.experimental.pallas import tpu_sc as plsc

N = 10000
E = 320000
D = 128

NC = 2
NS = 16
C = 128
CH = 80
CHS = 40
E_PAD = NC * NS * CH * C
N_PAD = 10240
ZROWS = N_PAD // NS

_sc_mesh = plsc.VectorSubcoreMesh(core_axis_name="c", subcore_axis_name="s")


@functools.partial(
    pl.kernel,
    out_type=jax.ShapeDtypeStruct((NC, N_PAD, D), jnp.float32),
    mesh=_sc_mesh,
    scratch_types=[
        pltpu.VMEM((CHS, C), jnp.int32),
        pltpu.VMEM((CHS, C), jnp.int32),
        pltpu.VMEM((C, D), jnp.float32),
        pltpu.VMEM((C, D), jnp.float32),
        pltpu.VMEM_SHARED((N_PAD, D), jnp.float32),
        pltpu.SemaphoreType.DMA,
        pltpu.SemaphoreType.DMA,
    ],
)
def _sc_aggregate(x_hbm, src_hbm, dst_hbm, out_hbm, src_v, dst_v, rows_a,
                  rows_b, acc_sh, sem_a, sem_b):
    c = lax.axis_index("c")
    s = lax.axis_index("s")

    zeros16 = jnp.zeros((16,), jnp.float32)

    def _zero_row(r, carry):
        for l in range(D // 16):
            rows_a[r, pl.ds(l * 16, 16)] = zeros16
        return carry

    lax.fori_loop(0, C, _zero_row, 0)
    for k in range(ZROWS // C):
        pltpu.sync_copy(rows_a, acc_sh.at[pl.ds(s * ZROWS + k * C, C)])

    plsc.subcore_barrier()

    for half in range(CH // CHS):
        pltpu.sync_copy(src_hbm.at[c, s, pl.ds(half * CHS, CHS)], src_v)
        pltpu.sync_copy(dst_hbm.at[c, s, pl.ds(half * CHS, CHS)], dst_v)

        pltpu.async_copy(x_hbm.at[src_v.at[0]], rows_a, sem_a)
        pltpu.async_copy(x_hbm.at[src_v.at[1]], rows_b, sem_b)

        def _pair(k, carry):
            g = 2 * k
            pltpu.make_async_copy(x_hbm.at[src_v.at[g]], rows_a,
                                  sem_a).wait()
            pltpu.sync_copy(rows_a, acc_sh.at[dst_v.at[g]], add=True)
            pltpu.async_copy(x_hbm.at[src_v.at[g + 2]], rows_a, sem_a)
            pltpu.make_async_copy(x_hbm.at[src_v.at[g + 1]], rows_b,
                                  sem_b).wait()
            pltpu.sync_copy(rows_b, acc_sh.at[dst_v.at[g + 1]], add=True)
            pltpu.async_copy(x_hbm.at[src_v.at[g + 3]], rows_b, sem_b)
            return carry

        lax.fori_loop(0, CHS // 2 - 1, _pair, 0)

        g_last = CHS - 2
        pltpu.make_async_copy(x_hbm.at[src_v.at[g_last]], rows_a,
                              sem_a).wait()
        pltpu.sync_copy(rows_a, acc_sh.at[dst_v.at[g_last]], add=True)
        pltpu.make_async_copy(x_hbm.at[src_v.at[g_last + 1]], rows_b,
                              sem_b).wait()
        pltpu.sync_copy(rows_b, acc_sh.at[dst_v.at[g_last + 1]], add=True)

    plsc.subcore_barrier()

    pltpu.sync_copy(acc_sh.at[pl.ds(s * ZROWS, ZROWS)],
                    out_hbm.at[c, pl.ds(s * ZROWS, ZROWS)])


def _tc_mlp_body(x_ref, p0_ref, p1_ref, eps_ref, w1_ref, b1_ref, w2_ref,
                 b2_ref, g_ref, be_ref, o_ref):
    h = (1.0 + eps_ref[0, 0]) * x_ref[...] + p0_ref[...] + p1_ref[...]
    h1 = lax.dot_general(h, w1_ref[...], (((1,), (1,)), ((), ())),
                         preferred_element_type=jnp.float32) + b1_ref[...]
    h1 = jnp.maximum(h1, 0.0)
    h2 = lax.dot_general(h1, w2_ref[...], (((1,), (1,)), ((), ())),
                         preferred_element_type=jnp.float32) + b2_ref[...]
    mean = jnp.mean(h2, axis=-1, keepdims=True)
    cent = h2 - mean
    var = jnp.mean(cent * cent, axis=-1, keepdims=True)
    o_ref[...] = cent * lax.rsqrt(var + 1e-5) * g_ref[...] + be_ref[...]


_TC_BLK = 1000


def _tc_mlp(x, p0, p1, eps, W1, b1, W2, b2, gamma, beta):
    grid = (N // _TC_BLK,)
    row_spec = pl.BlockSpec((_TC_BLK, D), lambda i: (i, 0))
    full_spec = pl.BlockSpec((D, D), lambda i: (0, 0))
    vec_spec = pl.BlockSpec((1, D), lambda i: (0, 0))
    return pl.pallas_call(
        _tc_mlp_body,
        grid=grid,
        in_specs=[
            row_spec, row_spec, row_spec,
            pl.BlockSpec((1, 1), lambda i: (0, 0)),
            full_spec, vec_spec, full_spec, vec_spec, vec_spec, vec_spec,
        ],
        out_specs=row_spec,
        out_shape=jax.ShapeDtypeStruct((N, D), jnp.float32),
    )(x, p0, p1, eps, W1, b1, W2, b2, gamma, beta)


def kernel(x, edge_index, eps, W1, b1, W2, b2, gamma, beta):
    dst = edge_index[0]
    src = edge_index[1]
    pad = E_PAD - E
    src_p = jnp.concatenate(
        [src, jnp.zeros((pad,), jnp.int32)]).reshape(NC, NS, CH, C)
    dst_p = jnp.concatenate(
        [dst, jnp.full((pad,), N, jnp.int32)]).reshape(NC, NS, CH, C)

    partials = _sc_aggregate(x, src_p, dst_p)

    eps2 = jnp.reshape(eps, (1, 1)).astype(jnp.float32)
    return _tc_mlp(x, partials[0], partials[1], eps2, W1,
                   jnp.reshape(b1, (1, D)), W2, jnp.reshape(b2, (1, D)),
                   jnp.reshape(gamma, (1, D)), jnp.reshape(beta, (1, D)))


# speedup vs baseline: 1.0006x; 1.0006x over previous
import functools

import jax
import jax.numpy as jnp
from jax import lax
from jax.experimental import pallas as pl
from jax.experimental.pallas import tpu as pltpu
from jax.experimental.pallas import tpu_sc as plsc

N = 10000
E = 320000
D = 128

NC = 2
NS = 16
C = 128
CH = 80
CHS = 40
E_PAD = NC * NS * CH * C
N_PAD = 10240
ZROWS = N_PAD // NS

_sc_mesh = plsc.VectorSubcoreMesh(core_axis_name="c", subcore_axis_name="s")


@functools.partial(
    pl.kernel,
    out_type=jax.ShapeDtypeStruct((NC, N_PAD, D), jnp.float32),
    mesh=_sc_mesh,
    scratch_types=[
        pltpu.VMEM((CHS, C), jnp.int32),
        pltpu.VMEM((CHS, C), jnp.int32),
        pltpu.VMEM((C, D), jnp.float32),
        pltpu.VMEM((C, D), jnp.float32),
        pltpu.VMEM_SHARED((N_PAD, D), jnp.float32),
        pltpu.SemaphoreType.DMA,
        pltpu.SemaphoreType.DMA,
    ],
)
def _sc_aggregate(x_hbm, src_hbm, dst_hbm, out_hbm, src_v, dst_v, rows_a,
                  rows_b, acc_sh, sem_a, sem_b):
    c = lax.axis_index("c")
    s = lax.axis_index("s")

    zeros16 = jnp.zeros((16,), jnp.float32)

    def _zero_row(r, carry):
        for l in range(D // 16):
            rows_a[r, pl.ds(l * 16, 16)] = zeros16
        return carry

    lax.fori_loop(0, C, _zero_row, 0)
    for k in range(ZROWS // C):
        pltpu.sync_copy(rows_a, acc_sh.at[pl.ds(s * ZROWS + k * C, C)])

    plsc.subcore_barrier()

    for half in range(CH // CHS):
        pltpu.sync_copy(src_hbm.at[c, s, pl.ds(half * CHS, CHS)], src_v)
        pltpu.sync_copy(dst_hbm.at[c, s, pl.ds(half * CHS, CHS)], dst_v)

        pltpu.async_copy(x_hbm.at[src_v.at[0]], rows_a, sem_a)
        pltpu.async_copy(x_hbm.at[src_v.at[1]], rows_b, sem_b)

        def _pair(k, carry):
            g = 2 * k
            pltpu.make_async_copy(x_hbm.at[src_v.at[g]], rows_a,
                                  sem_a).wait()
            pltpu.sync_copy(rows_a, acc_sh.at[dst_v.at[g]], add=True)
            pltpu.async_copy(x_hbm.at[src_v.at[g + 2]], rows_a, sem_a)
            pltpu.make_async_copy(x_hbm.at[src_v.at[g + 1]], rows_b,
                                  sem_b).wait()
            pltpu.sync_copy(rows_b, acc_sh.at[dst_v.at[g + 1]], add=True)
            pltpu.async_copy(x_hbm.at[src_v.at[g + 3]], rows_b, sem_b)
            return carry

        lax.fori_loop(0, CHS // 2 - 1, _pair, 0)

        g_last = CHS - 2
        pltpu.make_async_copy(x_hbm.at[src_v.at[g_last]], rows_a,
                              sem_a).wait()
        pltpu.sync_copy(rows_a, acc_sh.at[dst_v.at[g_last]], add=True)
        pltpu.make_async_copy(x_hbm.at[src_v.at[g_last + 1]], rows_b,
                              sem_b).wait()
        pltpu.sync_copy(rows_b, acc_sh.at[dst_v.at[g_last + 1]], add=True)

    plsc.subcore_barrier()

    pltpu.sync_copy(acc_sh.at[pl.ds(s * ZROWS, ZROWS)],
                    out_hbm.at[c, pl.ds(s * ZROWS, ZROWS)])


def _tc_mlp_body(x_ref, p0_ref, p1_ref, eps_ref, w1_ref, b1_ref, w2_ref,
                 b2_ref, g_ref, be_ref, o_ref):
    h = (1.0 + eps_ref[0, 0]) * x_ref[...] + p0_ref[...] + p1_ref[...]
    h1 = lax.dot_general(h, w1_ref[...], (((1,), (1,)), ((), ())),
                         preferred_element_type=jnp.float32) + b1_ref[...]
    h1 = jnp.maximum(h1, 0.0)
    h2 = lax.dot_general(h1, w2_ref[...], (((1,), (1,)), ((), ())),
                         preferred_element_type=jnp.float32) + b2_ref[...]
    mean = jnp.mean(h2, axis=-1, keepdims=True)
    cent = h2 - mean
    var = jnp.mean(cent * cent, axis=-1, keepdims=True)
    o_ref[...] = cent * lax.rsqrt(var + 1e-5) * g_ref[...] + be_ref[...]


_TC_BLK = 1000


def _tc_mlp(x, p0, p1, eps, W1, b1, W2, b2, gamma, beta):
    grid = (N // _TC_BLK,)
    row_spec = pl.BlockSpec((_TC_BLK, D), lambda i: (i, 0))
    full_spec = pl.BlockSpec((D, D), lambda i: (0, 0))
    vec_spec = pl.BlockSpec((1, D), lambda i: (0, 0))
    return pl.pallas_call(
        _tc_mlp_body,
        grid=grid,
        in_specs=[
            row_spec, row_spec, row_spec,
            pl.BlockSpec((1, 1), lambda i: (0, 0)),
            full_spec, vec_spec, full_spec, vec_spec, vec_spec, vec_spec,
        ],
        out_specs=row_spec,
        out_shape=jax.ShapeDtypeStruct((N, D), jnp.float32),
    )(x, p0, p1, eps, W1, b1, W2, b2, gamma, beta)


def kernel(x, edge_index, eps, W1, b1, W2, b2, gamma, beta):
    dst = edge_index[0]
    src = edge_index[1]
    pad = E_PAD - E
    src_p = jnp.concatenate(
        [src, jnp.zeros((pad,), jnp.int32)]).reshape(NC, NS, CH, C)
    pad_dst = N + jnp.arange(pad, dtype=jnp.int32) % (N_PAD - N)
    dst_p = jnp.concatenate([dst, pad_dst]).reshape(NC, NS, CH, C)

    partials = _sc_aggregate(x, src_p, dst_p)

    eps2 = jnp.reshape(eps, (1, 1)).astype(jnp.float32)
    return _tc_mlp(x, partials[0], partials[1], eps2, W1,
                   jnp.reshape(b1, (1, D)), W2, jnp.reshape(b2, (1, D)),
                   jnp.reshape(gamma, (1, D)), jnp.reshape(beta, (1, D)))


# speedup vs baseline: 3.4179x; 3.4158x over previous
import functools

import jax
import jax.numpy as jnp
from jax import lax
from jax.experimental import pallas as pl
from jax.experimental.pallas import tpu as pltpu
from jax.experimental.pallas import tpu_sc as plsc

N = 10000
E = 320000
D = 128

NC = 2
NS = 16
C = 128
CH = 80
CHS = 40
E_PAD = NC * NS * CH * C
N_PAD = 10240
ZROWS = N_PAD // NS

_sc_mesh = plsc.VectorSubcoreMesh(core_axis_name="c", subcore_axis_name="s")


@functools.partial(
    pl.kernel,
    out_type=jax.ShapeDtypeStruct((NC, N_PAD, D), jnp.float32),
    mesh=_sc_mesh,
    scratch_types=[
        pltpu.VMEM((CHS, C), jnp.int32),
        pltpu.VMEM((CHS, C), jnp.int32),
        pltpu.VMEM((C, D), jnp.float32),
        pltpu.VMEM((C, D), jnp.float32),
        pltpu.VMEM_SHARED((N_PAD, D), jnp.float32),
        pltpu.SemaphoreType.DMA,
        pltpu.SemaphoreType.DMA,
    ],
)
def _sc_aggregate(x_hbm, src_hbm, dst_hbm, out_hbm, src_v, dst_v, rows_a,
                  rows_b, acc_sh, sem_a, sem_b):
    c = lax.axis_index("c")
    s = lax.axis_index("s")

    zeros16 = jnp.zeros((16,), jnp.float32)

    def _zero_row(r, carry):
        for l in range(D // 16):
            rows_a[r, pl.ds(l * 16, 16)] = zeros16
        return carry

    lax.fori_loop(0, C, _zero_row, 0)
    for k in range(ZROWS // C):
        pltpu.sync_copy(rows_a, acc_sh.at[pl.ds(s * ZROWS + k * C, C)])

    plsc.subcore_barrier()

    for half in range(CH // CHS):
        pltpu.sync_copy(src_hbm.at[c, s, pl.ds(half * CHS, CHS)], src_v)
        pltpu.sync_copy(dst_hbm.at[c, s, pl.ds(half * CHS, CHS)], dst_v)

        pltpu.async_copy(x_hbm.at[src_v.at[0]], rows_a, sem_a)
        pltpu.async_copy(x_hbm.at[src_v.at[1]], rows_b, sem_b)

        def _pair(k, carry):
            g = 2 * k
            pltpu.make_async_copy(x_hbm.at[src_v.at[g]], rows_a,
                                  sem_a).wait()
            pltpu.sync_copy(rows_a, acc_sh.at[dst_v.at[g]], add=True)
            pltpu.async_copy(x_hbm.at[src_v.at[g + 2]], rows_a, sem_a)
            pltpu.make_async_copy(x_hbm.at[src_v.at[g + 1]], rows_b,
                                  sem_b).wait()
            pltpu.sync_copy(rows_b, acc_sh.at[dst_v.at[g + 1]], add=True)
            pltpu.async_copy(x_hbm.at[src_v.at[g + 3]], rows_b, sem_b)
            return carry

        lax.fori_loop(0, CHS // 2 - 1, _pair, 0)

        g_last = CHS - 2
        pltpu.make_async_copy(x_hbm.at[src_v.at[g_last]], rows_a,
                              sem_a).wait()
        pltpu.sync_copy(rows_a, acc_sh.at[dst_v.at[g_last]], add=True)
        pltpu.make_async_copy(x_hbm.at[src_v.at[g_last + 1]], rows_b,
                              sem_b).wait()
        pltpu.sync_copy(rows_b, acc_sh.at[dst_v.at[g_last + 1]], add=True)

    plsc.subcore_barrier()

    pltpu.sync_copy(acc_sh.at[pl.ds(s * ZROWS, ZROWS)],
                    out_hbm.at[c, pl.ds(s * ZROWS, ZROWS)])


def _tc_mlp_body(x_ref, p0_ref, p1_ref, eps_ref, w1_ref, b1_ref, w2_ref,
                 b2_ref, g_ref, be_ref, o_ref):
    h = (1.0 + eps_ref[0, 0]) * x_ref[...] + p0_ref[...] + p1_ref[...]
    h1 = lax.dot_general(h, w1_ref[...], (((1,), (1,)), ((), ())),
                         preferred_element_type=jnp.float32) + b1_ref[...]
    h1 = jnp.maximum(h1, 0.0)
    h2 = lax.dot_general(h1, w2_ref[...], (((1,), (1,)), ((), ())),
                         preferred_element_type=jnp.float32) + b2_ref[...]
    mean = jnp.mean(h2, axis=-1, keepdims=True)
    cent = h2 - mean
    var = jnp.mean(cent * cent, axis=-1, keepdims=True)
    o_ref[...] = cent * lax.rsqrt(var + 1e-5) * g_ref[...] + be_ref[...]


_TC_BLK = 1000


def _tc_mlp(x, p0, p1, eps, W1, b1, W2, b2, gamma, beta):
    grid = (N // _TC_BLK,)
    row_spec = pl.BlockSpec((_TC_BLK, D), lambda i: (i, 0))
    full_spec = pl.BlockSpec((D, D), lambda i: (0, 0))
    vec_spec = pl.BlockSpec((1, D), lambda i: (0, 0))
    return pl.pallas_call(
        _tc_mlp_body,
        grid=grid,
        in_specs=[
            row_spec, row_spec, row_spec,
            pl.BlockSpec((1, 1), lambda i: (0, 0)),
            full_spec, vec_spec, full_spec, vec_spec, vec_spec, vec_spec,
        ],
        out_specs=row_spec,
        out_shape=jax.ShapeDtypeStruct((N, D), jnp.float32),
    )(x, p0, p1, eps, W1, b1, W2, b2, gamma, beta)


def kernel(x, edge_index, eps, W1, b1, W2, b2, gamma, beta):
    dst = edge_index[0]
    src = edge_index[1]
    pad = E_PAD - E
    pad_src = jnp.arange(pad, dtype=jnp.int32) % N
    src_p = jnp.concatenate([src, pad_src]).reshape(NC, NS, CH, C)
    pad_dst = N + jnp.arange(pad, dtype=jnp.int32) % (N_PAD - N)
    dst_p = jnp.concatenate([dst, pad_dst]).reshape(NC, NS, CH, C)

    partials = _sc_aggregate(x, src_p, dst_p)

    eps2 = jnp.reshape(eps, (1, 1)).astype(jnp.float32)
    return _tc_mlp(x, partials[0], partials[1], eps2, W1,
                   jnp.reshape(b1, (1, D)), W2, jnp.reshape(b2, (1, D)),
                   jnp.reshape(gamma, (1, D)), jnp.reshape(beta, (1, D)))
